# baseline (speedup 1.0000x reference)
import functools

import jax
import jax.numpy as jnp
from jax import lax
from jax.experimental import pallas as pl
from jax.experimental.pallas import tpu as pltpu
from jax.experimental.pallas import tpu_sc as plsc
from jax.experimental.layout import Layout, with_layout_constraint

B = 64
N = 262144
C = 2

_NC = 2
_NS = 16
_NW = _NC * _NS
_RPW = N // _NW
_TB = N // 128
_G = 256
_NCH = _RPW // _G
_ROWW = C * B
_BSTR = N * C

_mesh = plsc.VectorSubcoreMesh(core_axis_name="c", subcore_axis_name="s")
_params = pltpu.CompilerParams(needs_layout_passes=False)


_OFFS = [8192 * (k % 4) + 128 * (k // 4) for k in range(8)]
_SLC = _G * _ROWW - max(_OFFS)


def _pattern():
    return lax.iota(jnp.int32, 16) * 512


@functools.partial(
    pl.kernel,
    mesh=_mesh,
    compiler_params=_params,
    out_type=jax.ShapeDtypeStruct((N, _ROWW), jnp.float32),
    scratch_types=[
        pltpu.VMEM((_G * _ROWW,), jnp.float32),
        pltpu.VMEM((_G * _ROWW,), jnp.float32),
        pltpu.VMEM((_G, _ROWW), jnp.float32),
        pltpu.SemaphoreType.DMA,
        pltpu.SemaphoreType.DMA,
        pltpu.SemaphoreType.DMA,
    ],
)
def _transpose_kernel(x_hbm, xt_hbm, in0_v, in1_v, tile_v, lsem0, lsem1, wsem):
    wid = lax.axis_index("s") * _NC + lax.axis_index("c")
    pat = _pattern()
    ins = (in0_v, in1_v)
    lsems = (lsem0, lsem1)

    def src_slice(k, b):
        start = b * _BSTR + (wid * 128 + 4 * k) * 128
        return x_hbm.at[pl.ds(start, 4 * 128)]

    def loads_start(k, p):
        for b in range(B):
            pltpu.async_copy(src_slice(k, b), ins[p].at[pl.ds(b * 512, 512)], lsems[p])

    def loads_wait(k, p):
        for b in range(B):
            pltpu.make_async_copy(
                src_slice(k, b), ins[p].at[pl.ds(b * 512, 512)], lsems[p]
            ).wait()

    def compute(k, p):
        buf = ins[p]

        def half(dt):
            @plsc.parallel_loop(0, 128, unroll=4)
            def _(u):
                idx = pat + (256 * dt + u)
                vs = [
                    plsc.load_gather(buf.at[pl.ds(_OFFS[kk], _SLC)], [idx])
                    for kk in range(8)
                ]
                for kk in range(8):
                    tile_v[128 * dt + u, pl.ds(16 * kk, 16)] = vs[kk]

        half(0)
        half(1)

    def write(k):
        pltpu.async_copy(tile_v, xt_hbm.at[pl.ds(wid * _RPW + k * _G, _G), :], wsem)

    def write_wait(k):
        pltpu.make_async_copy(
            tile_v, xt_hbm.at[pl.ds(wid * _RPW + k * _G, _G), :], wsem
        ).wait()

    loads_start(0, 0)

    def body(m, carry):
        p = 0
        k = 2 * m
        loads_wait(k, 0)
        loads_start(k + 1, 1)
        compute(k, 0)
        write(k)
        loads_wait(k + 1, 1)

        @pl.when(m + 1 < _NCH // 2)
        def _():
            loads_start(k + 2, 0)

        write_wait(k)
        compute(k + 1, 1)
        write(k + 1)
        write_wait(k + 1)
        return carry

    lax.fori_loop(0, _NCH // 2, body, 0)


@functools.partial(
    pl.kernel,
    mesh=_mesh,
    compiler_params=_params,
    out_type=jax.ShapeDtypeStruct((B * N * C,), jnp.float32),
    scratch_types=[
        pltpu.VMEM((_RPW,), jnp.int32),
        pltpu.VMEM((_G, _ROWW), jnp.float32),
        pltpu.VMEM((_G, _ROWW), jnp.float32),
        pltpu.VMEM((_G * _ROWW,), jnp.float32),
        pltpu.SemaphoreType.DMA,
        pltpu.SemaphoreType.DMA,
        pltpu.SemaphoreType.DMA,
    ],
)
def _gather_kernel(
    xt_hbm, perm_hbm, out_hbm, idx_v, rows0_v, rows1_v, out_v, gsem0, gsem1, wsem
):
    wid = lax.axis_index("s") * _NC + lax.axis_index("c")
    base = wid * _RPW
    pltpu.sync_copy(perm_hbm.at[pl.ds(base, _RPW)], idx_v)
    pat = _pattern()
    rows = (rows0_v, rows1_v)
    gsems = (gsem0, gsem1)

    def gather_start(k, p):
        pltpu.async_copy(
            xt_hbm.at[idx_v.at[pl.ds(k * _G, _G)]], rows[p], gsems[p]
        )

    def gather_wait(k, p):
        pltpu.make_async_copy(
            xt_hbm.at[idx_v.at[pl.ds(k * _G, _G)]], rows[p], gsems[p]
        ).wait()

    def compute(k, p):
        buf = rows[p]

        def half(dt):
            @plsc.parallel_loop(0, 128, unroll=4)
            def _(u):
                idx = pat + (256 * dt + u)
                vs = [buf[128 * dt + u, pl.ds(16 * kk, 16)] for kk in range(8)]
                for kk in range(8):
                    plsc.store_scatter(
                        out_v.at[pl.ds(_OFFS[kk], _SLC)], [idx], vs[kk]
                    )

        half(0)
        half(1)

    def writes_start(k):
        for b in range(B):
            start = b * _BSTR + (wid * 128 + 4 * k) * 128
            pltpu.async_copy(
                out_v.at[pl.ds(b * 512, 512)], out_hbm.at[pl.ds(start, 512)], wsem
            )

    def writes_wait(k):
        for b in range(B):
            start = b * _BSTR + (wid * 128 + 4 * k) * 128
            pltpu.make_async_copy(
                out_v.at[pl.ds(b * 512, 512)], out_hbm.at[pl.ds(start, 512)], wsem
            ).wait()

    gather_start(0, 0)

    def body(m, carry):
        k = 2 * m
        gather_start(k + 1, 1)
        gather_wait(k, 0)
        compute(k, 0)
        writes_start(k)

        @pl.when(m + 1 < _NCH // 2)
        def _():
            gather_start(k + 2, 0)

        gather_wait(k + 1, 1)
        writes_wait(k)
        compute(k + 1, 1)
        writes_start(k + 1)
        writes_wait(k + 1)
        return carry

    lax.fori_loop(0, _NCH // 2, body, 0)


def _raw_in_view(x):
    x4 = x.reshape(B, _TB, 128, C)
    x4 = with_layout_constraint(
        x4, Layout(major_to_minor=(0, 1, 3, 2), tiling=((2, 128),))
    )
    x5 = jnp.transpose(x4, (0, 1, 3, 2))
    x5 = with_layout_constraint(
        x5, Layout(major_to_minor=(0, 1, 2, 3), tiling=((2, 128),))
    )
    return x5.reshape(B * N * C)


def _raw_out_view(o):
    o5 = o.reshape(B, _TB, C, 128)
    o5 = with_layout_constraint(
        o5, Layout(major_to_minor=(0, 1, 2, 3), tiling=((2, 128),))
    )
    o4 = jnp.transpose(o5, (0, 1, 3, 2))
    o4 = with_layout_constraint(
        o4, Layout(major_to_minor=(0, 1, 3, 2), tiling=((2, 128),))
    )
    return o4.reshape(B, N, C)


def kernel(x, perm):
    xt = _transpose_kernel(_raw_in_view(x))
    return _raw_out_view(_gather_kernel(xt, perm))


# speedup vs baseline: 4.1615x; 4.1615x over previous
import functools

import jax
import jax.numpy as jnp
from jax import lax
from jax.experimental import pallas as pl
from jax.experimental.pallas import tpu as pltpu
from jax.experimental.pallas import tpu_sc as plsc
from jax.experimental.layout import Layout, with_layout_constraint

B = 64
N = 262144
C = 2

_NC = 2
_NS = 16
_NW = _NC * _NS
_RPW = N // _NW
_TB = N // 128
_G = 256
_NCH = _RPW // _G
_ROWW = C * B
_BSTR = N * C

_mesh = plsc.VectorSubcoreMesh(core_axis_name="c", subcore_axis_name="s")
_params = pltpu.CompilerParams(needs_layout_passes=False)


def _rots():
    lanes = lax.iota(jnp.int32, 16)
    return lanes, [
        lax.rem(lanes + d, jnp.int32(16)) if d else lanes for d in range(16)
    ]


@functools.partial(
    pl.kernel,
    mesh=_mesh,
    compiler_params=_params,
    out_type=jax.ShapeDtypeStruct((N * _ROWW,), jnp.float32),
    scratch_types=[
        pltpu.VMEM((_G * _ROWW,), jnp.float32),
        pltpu.VMEM((_G * _ROWW,), jnp.float32),
        pltpu.VMEM((_G * _ROWW,), jnp.float32),
        pltpu.SemaphoreType.DMA,
        pltpu.SemaphoreType.DMA,
        pltpu.SemaphoreType.DMA,
    ],
)
def _transpose_kernel(x_hbm, xt_hbm, in0_v, in1_v, tile_v, lsem0, lsem1, wsem):
    wid = lax.axis_index("s") * _NC + lax.axis_index("c")
    lanes, rots = _rots()
    lanes128 = lanes * 128
    ins = (in0_v, in1_v)
    lsems = (lsem0, lsem1)

    def src_slice(k, b):
        start = b * _BSTR + (wid * 128 + 4 * k) * 128
        return x_hbm.at[pl.ds(start, 4 * 128)]

    def loads_start(k, p):
        for b in range(B):
            pltpu.async_copy(src_slice(k, b), ins[p].at[pl.ds(b * 512, 512)], lsems[p])

    def loads_wait(k, p):
        for b in range(B):
            pltpu.make_async_copy(
                src_slice(k, b), ins[p].at[pl.ds(b * 512, 512)], lsems[p]
            ).wait()

    def compute(k, p):
        buf = ins[p]

        @plsc.parallel_loop(0, 128, unroll=2)
        def _(m):
            u0 = (m & 7) * 16
            b0 = ((m >> 3) & 3) * 16
            c = (m >> 5) & 1
            dt = m >> 6
            ivec = lanes + (512 * b0 + 128 * (2 * dt + c) + u0)
            tvec = lanes128 + (16384 * dt + 64 * c + b0 + 128 * u0)
            for d in range(16):
                v = plsc.load_gather(buf, [(rots[d] << 9) + ivec])
                plsc.store_scatter(tile_v, [rots[d] + tvec], v)

    def write(k):
        pltpu.async_copy(
            tile_v, xt_hbm.at[pl.ds((wid * _RPW + k * _G) * _ROWW, _G * _ROWW)], wsem
        )

    def write_wait(k):
        pltpu.make_async_copy(
            tile_v, xt_hbm.at[pl.ds((wid * _RPW + k * _G) * _ROWW, _G * _ROWW)], wsem
        ).wait()

    loads_start(0, 0)

    def body(m, carry):
        p = 0
        k = 2 * m
        loads_wait(k, 0)
        loads_start(k + 1, 1)
        compute(k, 0)
        write(k)
        loads_wait(k + 1, 1)

        @pl.when(m + 1 < _NCH // 2)
        def _():
            loads_start(k + 2, 0)

        write_wait(k)
        compute(k + 1, 1)
        write(k + 1)
        write_wait(k + 1)
        return carry

    lax.fori_loop(0, _NCH // 2, body, 0)


@functools.partial(
    pl.kernel,
    mesh=_mesh,
    compiler_params=_params,
    out_type=jax.ShapeDtypeStruct((B * N * C,), jnp.float32),
    scratch_types=[
        pltpu.VMEM((_RPW,), jnp.int32),
        pltpu.VMEM((_G, _ROWW), jnp.float32),
        pltpu.VMEM((_G, _ROWW), jnp.float32),
        pltpu.VMEM((_G * _ROWW,), jnp.float32),
        pltpu.SemaphoreType.DMA,
        pltpu.SemaphoreType.DMA,
        pltpu.SemaphoreType.DMA,
    ],
)
def _gather_kernel(
    xt_hbm, perm_hbm, out_hbm, idx_v, rows0_v, rows1_v, out_v, gsem0, gsem1, wsem
):
    wid = lax.axis_index("s") * _NC + lax.axis_index("c")
    base = wid * _RPW
    pltpu.sync_copy(perm_hbm.at[pl.ds(base, _RPW)], idx_v)
    lanes, rots = _rots()
    rows = (rows0_v, rows1_v)
    gsems = (gsem0, gsem1)

    def gather_start(k, p):
        pltpu.async_copy(
            xt_hbm.at[idx_v.at[pl.ds(k * _G, _G)]], rows[p], gsems[p]
        )

    def gather_wait(k, p):
        pltpu.make_async_copy(
            xt_hbm.at[idx_v.at[pl.ds(k * _G, _G)]], rows[p], gsems[p]
        ).wait()

    def compute(k, p):
        buf = rows[p]

        @plsc.parallel_loop(0, 128, unroll=2)
        def _(m):
            u0 = (m & 7) * 16
            b0 = ((m >> 3) & 3) * 16
            c = (m >> 5) & 1
            dt = m >> 6
            rvec = lanes + (128 * dt + u0)
            cbase = 64 * c + b0
            ovec = lanes + (512 * b0 + 128 * (2 * dt + c) + u0)
            for d in range(16):
                v = plsc.load_gather(buf, [rvec, rots[d] + cbase])
                plsc.store_scatter(out_v, [(rots[d] << 9) + ovec], v)

    def writes_start(k):
        for b in range(B):
            start = b * _BSTR + (wid * 128 + 4 * k) * 128
            pltpu.async_copy(
                out_v.at[pl.ds(b * 512, 512)], out_hbm.at[pl.ds(start, 512)], wsem
            )

    def writes_wait(k):
        for b in range(B):
            start = b * _BSTR + (wid * 128 + 4 * k) * 128
            pltpu.make_async_copy(
                out_v.at[pl.ds(b * 512, 512)], out_hbm.at[pl.ds(start, 512)], wsem
            ).wait()

    gather_start(0, 0)

    def body(m, carry):
        k = 2 * m
        gather_start(k + 1, 1)
        gather_wait(k, 0)
        compute(k, 0)
        writes_start(k)

        @pl.when(m + 1 < _NCH // 2)
        def _():
            gather_start(k + 2, 0)

        gather_wait(k + 1, 1)
        writes_wait(k)
        compute(k + 1, 1)
        writes_start(k + 1)
        writes_wait(k + 1)
        return carry

    lax.fori_loop(0, _NCH // 2, body, 0)


def _raw_in_view(x):
    x4 = x.reshape(B, _TB, 128, C)
    x4 = with_layout_constraint(
        x4, Layout(major_to_minor=(0, 1, 3, 2), tiling=((2, 128),))
    )
    x5 = jnp.transpose(x4, (0, 1, 3, 2))
    x5 = with_layout_constraint(
        x5, Layout(major_to_minor=(0, 1, 2, 3), tiling=((2, 128),))
    )
    return x5.reshape(B * N * C)


def _raw_out_view(o):
    o5 = o.reshape(B, _TB, C, 128)
    o5 = with_layout_constraint(
        o5, Layout(major_to_minor=(0, 1, 2, 3), tiling=((2, 128),))
    )
    o4 = jnp.transpose(o5, (0, 1, 3, 2))
    o4 = with_layout_constraint(
        o4, Layout(major_to_minor=(0, 1, 3, 2), tiling=((2, 128),))
    )
    return o4.reshape(B, N, C)


def kernel(x, perm):
    xt = _transpose_kernel(_raw_in_view(x)).reshape(N, _ROWW)
    return _raw_out_view(_gather_kernel(xt, perm))
